# Initial kernel scaffold; baseline (speedup 1.0000x reference)
import jax
import jax.numpy as jnp
from jax.experimental import pallas as pl


def kernel(inputs, embeddings):
    raise NotImplementedError("write your pallas kernel here")


# speedup vs baseline: 2.0351x; 2.0351x over previous
import functools

import jax
import jax.numpy as jnp
from jax.experimental import pallas as pl
from jax.experimental.pallas import tpu as pltpu

NUM_EMB = 1024
DIM = 64
BM = 512


def _vq_block(x_ref, e_ref, et_ref, o_ref):
    x = x_ref[...]
    e = e_ref[...]
    scores = jax.lax.dot_general(
        x, e, (((1,), (0,)), ((), ())), preferred_element_type=jnp.float32
    )
    esq = jnp.sum(e * e, axis=0, keepdims=True)
    d = esq - 2.0 * scores
    dmin = jnp.min(d, axis=1, keepdims=True)
    col = jax.lax.broadcasted_iota(jnp.int32, (BM, NUM_EMB), 1)
    idx = jnp.min(jnp.where(d <= dmin, col, NUM_EMB), axis=1, keepdims=True)
    onehot = (col == idx).astype(jnp.float32)
    o_ref[...] = jax.lax.dot_general(
        onehot, et_ref[...], (((1,), (0,)), ((), ())),
        preferred_element_type=jnp.float32,
    )


@jax.jit
def kernel(inputs, embeddings):
    flat = inputs.reshape(-1, inputs.shape[-1])
    m = flat.shape[0]
    et = embeddings.T
    out = pl.pallas_call(
        _vq_block,
        grid=(m // BM,),
        in_specs=[
            pl.BlockSpec((BM, DIM), lambda i: (i, 0)),
            pl.BlockSpec((DIM, NUM_EMB), lambda i: (0, 0)),
            pl.BlockSpec((NUM_EMB, DIM), lambda i: (0, 0)),
        ],
        out_specs=pl.BlockSpec((BM, DIM), lambda i: (i, 0)),
        out_shape=jax.ShapeDtypeStruct((m, DIM), jnp.float32),
        compiler_params=pltpu.CompilerParams(
            dimension_semantics=("arbitrary",),
        ),
    )(flat, embeddings, et)
    return out.reshape(inputs.shape)
